# baseline (speedup 1.0000x reference)
import jax
import jax.numpy as jnp
from jax import lax
from jax.experimental import pallas as pl
from jax.experimental.pallas import tpu as pltpu
from jax.experimental.pallas import tpu_sc as plsc

VOCAB = 100000
D = 320
DP = 384
K = 20
NUM_CHUNKS = 50
B = 128
R = NUM_CHUNKS * B
NW = 32
ROWS_PER_W = R // NW
G = 4
NB = ROWS_PER_W // G
LANES = 16
NVD = D // LANES
NBLK = VOCAB // 128
VTAIL = VOCAB - NBLK * 128
BLK_SLOTS = 13


def _wid():
    return lax.axis_index("s") * 2 + lax.axis_index("c")


def _transpose_kernel(tabT, tail32, tabp, in0, in1, ob0, ob1, isem0, isem1,
                      osem0, osem1):
    wid = _wid()
    lanes = lax.iota(jnp.int32, 16)
    cmods = [(lanes + k) & 15 for k in range(16)]
    inbufs = (in0, in1)
    isems = (isem0, isem1)
    obufs = (ob0, ob1)
    osems = (osem0, osem1)

    NQ = 4

    def fire_in(j, u):
        for q in range(NQ):
            rq = pl.ds(q * (D // NQ), D // NQ)
            pltpu.async_copy(tabT.at[rq, pl.ds(j * 128, 128)],
                             inbufs[u].at[rq], isems[u])

    def drain_in(j, u):
        for q in range(NQ):
            rq = pl.ds(q * (D // NQ), D // NQ)
            pltpu.make_async_copy(tabT.at[rq, pl.ds(j * 128, 128)],
                                  inbufs[u].at[rq], isems[u]).wait()

    fire_in(wid, 0)

    def transpose_chunk(inbuf, h, hpar, j, t):
        @pl.when(t * 4 + h >= 2)
        def _():
            pltpu.make_async_copy(
                obufs[hpar],
                tabp.at[pl.ds(0, 32), :],
                osems[hpar],
            ).wait()

        @plsc.parallel_loop(0, 40, unroll=2)
        def tbody(tl):
            rt = tl // NVD
            ct = tl % NVD
            c0 = ct * 16
            src_v = h * 32 + rt * 16 + lanes
            dst_r = rt * 16 + lanes
            for k in range(16):
                cidx = c0 + cmods[k]
                v = plsc.load_gather(inbuf, [cidx, src_v])
                plsc.store_scatter(obufs[hpar], [dst_r, cidx], v)

        pltpu.async_copy(
            obufs[hpar],
            tabp.at[pl.ds(j * 128 + h * 32, 32), :],
            osems[hpar],
        )

    def body(tt, carry):
        for u in range(2):
            t = tt * 2 + u
            j = wid + t * NW

            @pl.when(j < NBLK)
            def _():
                drain_in(j, u)
                jn = j + NW

                @pl.when(jn < NBLK)
                def _():
                    fire_in(jn, 1 - u)

                for h in range(4):
                    transpose_chunk(inbufs[u], h, h % 2, j, t)

        return carry

    lax.fori_loop(0, BLK_SLOTS, body, 0)

    for hpar in range(2):
        pltpu.make_async_copy(
            obufs[hpar],
            tabp.at[pl.ds(0, 32), :],
            osems[hpar],
        ).wait()

    @pl.when(wid == NW - 1)
    def _():
        pltpu.sync_copy(tail32, obufs[0])
        pltpu.sync_copy(obufs[0], tabp.at[pl.ds(NBLK * 128, VTAIL), :])


def _gather_kernel(idx_hbm, tabp, out_hbm, idx_v, gA0, gB0, gA1, gB1, gA2,
                   gB2, acc0, acc1, acc2, gsem0, gsem1, gsem2, osem0, osem1,
                   osem2):
    wid = _wid()
    base = wid * ROWS_PER_W
    gbufs = ((gA0, gB0), (gA1, gB1), (gA2, gB2))
    accs = (acc0, acc1, acc2)
    gsems = (gsem0, gsem1, gsem2)
    osems = (osem0, osem1, osem2)

    pltpu.sync_copy(idx_hbm.at[wid], idx_v)

    def fire(b, slot):
        idxs = idx_v.at[b, pl.ds(0, G * K)]
        pltpu.async_copy(tabp.at[idxs, pl.ds(0, 128)],
                         gbufs[slot][0].at[:, pl.ds(0, 128)], gsems[slot])
        pltpu.async_copy(tabp.at[idxs, pl.ds(128, 128)],
                         gbufs[slot][0].at[:, pl.ds(128, 128)], gsems[slot])
        pltpu.async_copy(tabp.at[idxs, pl.ds(256, 128)], gbufs[slot][1],
                         gsems[slot])

    def drain(b, slot):
        idxs = idx_v.at[b, pl.ds(0, G * K)]
        pltpu.make_async_copy(tabp.at[idxs, pl.ds(0, 128)],
                              gbufs[slot][0].at[:, pl.ds(0, 128)],
                              gsems[slot]).wait()
        pltpu.make_async_copy(tabp.at[idxs, pl.ds(128, 128)],
                              gbufs[slot][0].at[:, pl.ds(128, 128)],
                              gsems[slot]).wait()
        pltpu.make_async_copy(tabp.at[idxs, pl.ds(256, 128)], gbufs[slot][1],
                              gsems[slot]).wait()

    def reduce_batch(slot, aslot, arow0):
        def gbody(g, carry):
            for part, nv in ((0, 16), (1, 4)):
                @plsc.parallel_loop(0, nv, unroll=2)
                def vbody(vv):
                    lc = vv * 16
                    a = gbufs[slot][part][g * K, pl.ds(lc, 16)]
                    for p in range(1, K):
                        a = a + gbufs[slot][part][g * K + p, pl.ds(lc, 16)]
                    accs[aslot][arow0 + g, pl.ds(part * 256 + lc, 16)] = (
                        a * (1.0 / K))
            return carry

        lax.fori_loop(0, G, gbody, 0)

    def write_group(grp, aslot):
        pltpu.async_copy(accs[aslot], out_hbm.at[pl.ds(base + grp * 8, 8), :],
                         osems[aslot])

    def wait_group(aslot):
        pltpu.make_async_copy(accs[aslot], out_hbm.at[pl.ds(0, 8), :],
                              osems[aslot]).wait()

    fire(0, 0)
    fire(1, 1)
    fire(2, 2)

    def body(jj, carry):
        for u in range(6):
            b = jj * 6 + u
            slot = u % 3
            aslot = u // 2
            grp = jj * 3 + aslot
            drain(b, slot)

            if u % 2 == 0:
                @pl.when(grp >= 3)
                def _():
                    wait_group(aslot)

            reduce_batch(slot, aslot, (u % 2) * G)

            @pl.when(b + 3 < NB)
            def _():
                fire(b + 3, slot)
            if u % 2 == 1:
                write_group(grp, aslot)

        return carry

    lax.fori_loop(0, NB // 6, body, 0)

    drain(NB - 2, 0)
    wait_group(0)
    reduce_batch(0, 0, 0)
    drain(NB - 1, 1)
    reduce_batch(1, 0, G)
    write_group(NB // 2 - 1, 0)
    for aslot in range(3):
        wait_group(aslot)


@jax.jit
def kernel(tokens, table):
    tabT = table.T
    tail32 = jnp.pad(table[NBLK * 128:, :], ((0, 0), (0, DP - D)))
    idx = (
        tokens.astype(jnp.int32)
        .reshape(NUM_CHUNKS, K, B)
        .transpose(0, 2, 1)
        .reshape(NW, NB, G * K)
    )
    idx = jnp.pad(idx, ((0, 0), (0, 0), (0, 128 - G * K)))

    mesh = plsc.VectorSubcoreMesh(core_axis_name="c", subcore_axis_name="s")
    cp = pltpu.CompilerParams(use_tc_tiling_on_sc=True, needs_layout_passes=False)

    tabp = pl.kernel(
        _transpose_kernel,
        out_type=jax.ShapeDtypeStruct((VOCAB, DP), jnp.float32),
        mesh=mesh,
        scratch_types=[
            pltpu.VMEM((D, 128), jnp.float32),
            pltpu.VMEM((D, 128), jnp.float32),
        ] + [pltpu.VMEM((32, DP), jnp.float32)] * 2 + [
            pltpu.SemaphoreType.DMA,
            pltpu.SemaphoreType.DMA,
            pltpu.SemaphoreType.DMA,
            pltpu.SemaphoreType.DMA,
        ],
        compiler_params=cp,
    )(tabT, tail32)

    out = pl.kernel(
        _gather_kernel,
        out_type=jax.ShapeDtypeStruct((R, DP), jnp.float32),
        mesh=mesh,
        scratch_types=[pltpu.VMEM((NB, 128), jnp.int32)]
        + [pltpu.VMEM((G * K, 256), jnp.float32),
           pltpu.VMEM((G * K, 128), jnp.float32)] * 3
        + [pltpu.VMEM((8, DP), jnp.float32)] * 3
        + [pltpu.SemaphoreType.DMA] * 6,
        compiler_params=cp,
    )(idx, tabp)

    return out[:, :D].reshape(NUM_CHUNKS, B, D)


# speedup vs baseline: 1.0029x; 1.0029x over previous
import jax
import jax.numpy as jnp
from jax import lax
from jax.experimental import pallas as pl
from jax.experimental.pallas import tpu as pltpu
from jax.experimental.pallas import tpu_sc as plsc

VOCAB = 100000
D = 320
DP = 384
K = 20
NUM_CHUNKS = 50
B = 128
R = NUM_CHUNKS * B
NW = 32
ROWS_PER_W = R // NW
G = 4
NB = ROWS_PER_W // G
LANES = 16
NVD = D // LANES
NBLK = VOCAB // 128
VTAIL = VOCAB - NBLK * 128
BLK_SLOTS = 13


def _wid():
    return lax.axis_index("s") * 2 + lax.axis_index("c")


def _transpose_kernel(tabT, tail32, tabp, in0, in1, ob0, ob1, isem0, isem1,
                      osem0, osem1):
    wid = _wid()
    lanes = lax.iota(jnp.int32, 16)
    cmods = [(lanes + k) & 15 for k in range(16)]
    inbufs = (in0, in1)
    isems = (isem0, isem1)
    obufs = (ob0, ob1)
    osems = (osem0, osem1)

    NQ = 8

    def fire_in(j, u):
        for q in range(NQ):
            rq = pl.ds(q * (D // NQ), D // NQ)
            pltpu.async_copy(tabT.at[rq, pl.ds(j * 128, 128)],
                             inbufs[u].at[rq], isems[u])

    def drain_in(j, u):
        for q in range(NQ):
            rq = pl.ds(q * (D // NQ), D // NQ)
            pltpu.make_async_copy(tabT.at[rq, pl.ds(j * 128, 128)],
                                  inbufs[u].at[rq], isems[u]).wait()

    fire_in(wid, 0)

    def transpose_chunk(inbuf, h, hpar, j, t):
        @pl.when(t * 4 + h >= 2)
        def _():
            pltpu.make_async_copy(
                obufs[hpar],
                tabp.at[pl.ds(0, 32), :],
                osems[hpar],
            ).wait()

        @plsc.parallel_loop(0, 40, unroll=2)
        def tbody(tl):
            rt = tl // NVD
            ct = tl % NVD
            c0 = ct * 16
            src_v = h * 32 + rt * 16 + lanes
            dst_r = rt * 16 + lanes
            for k in range(16):
                cidx = c0 + cmods[k]
                v = plsc.load_gather(inbuf, [cidx, src_v])
                plsc.store_scatter(obufs[hpar], [dst_r, cidx], v)

        pltpu.async_copy(
            obufs[hpar],
            tabp.at[pl.ds(j * 128 + h * 32, 32), :],
            osems[hpar],
        )

    def body(tt, carry):
        for u in range(2):
            t = tt * 2 + u
            j = wid + t * NW

            @pl.when(j < NBLK)
            def _():
                drain_in(j, u)
                jn = j + NW

                @pl.when(jn < NBLK)
                def _():
                    fire_in(jn, 1 - u)

                for h in range(4):
                    transpose_chunk(inbufs[u], h, h % 2, j, t)

        return carry

    lax.fori_loop(0, BLK_SLOTS, body, 0)

    for hpar in range(2):
        pltpu.make_async_copy(
            obufs[hpar],
            tabp.at[pl.ds(0, 32), :],
            osems[hpar],
        ).wait()

    @pl.when(wid == NW - 1)
    def _():
        pltpu.sync_copy(tail32, obufs[0])
        pltpu.sync_copy(obufs[0], tabp.at[pl.ds(NBLK * 128, VTAIL), :])


def _gather_kernel(idx_hbm, tabp, out_hbm, idx_v, gA0, gB0, gA1, gB1, gA2,
                   gB2, acc0, acc1, acc2, gsem0, gsem1, gsem2, osem0, osem1,
                   osem2):
    wid = _wid()
    base = wid * ROWS_PER_W
    gbufs = ((gA0, gB0), (gA1, gB1), (gA2, gB2))
    accs = (acc0, acc1, acc2)
    gsems = (gsem0, gsem1, gsem2)
    osems = (osem0, osem1, osem2)

    pltpu.sync_copy(idx_hbm.at[wid], idx_v)

    def fire(b, slot):
        idxs = idx_v.at[b, pl.ds(0, G * K)]
        pltpu.async_copy(tabp.at[idxs, pl.ds(0, 256)], gbufs[slot][0],
                         gsems[slot])
        pltpu.async_copy(tabp.at[idxs, pl.ds(256, 128)], gbufs[slot][1],
                         gsems[slot])

    def drain(b, slot):
        idxs = idx_v.at[b, pl.ds(0, G * K)]
        pltpu.make_async_copy(tabp.at[idxs, pl.ds(0, 256)], gbufs[slot][0],
                              gsems[slot]).wait()
        pltpu.make_async_copy(tabp.at[idxs, pl.ds(256, 128)], gbufs[slot][1],
                              gsems[slot]).wait()

    def reduce_batch(slot, aslot, arow0):
        def gbody(g, carry):
            for part, nv in ((0, 16), (1, 4)):
                @plsc.parallel_loop(0, nv, unroll=2)
                def vbody(vv):
                    lc = vv * 16
                    a = gbufs[slot][part][g * K, pl.ds(lc, 16)]
                    for p in range(1, K):
                        a = a + gbufs[slot][part][g * K + p, pl.ds(lc, 16)]
                    accs[aslot][arow0 + g, pl.ds(part * 256 + lc, 16)] = (
                        a * (1.0 / K))
            return carry

        lax.fori_loop(0, G, gbody, 0)

    def write_group(grp, aslot):
        pltpu.async_copy(accs[aslot], out_hbm.at[pl.ds(base + grp * 8, 8), :],
                         osems[aslot])

    def wait_group(aslot):
        pltpu.make_async_copy(accs[aslot], out_hbm.at[pl.ds(0, 8), :],
                              osems[aslot]).wait()

    fire(0, 0)
    fire(1, 1)
    fire(2, 2)

    def body(jj, carry):
        for u in range(6):
            b = jj * 6 + u
            slot = u % 3
            aslot = u // 2
            grp = jj * 3 + aslot
            drain(b, slot)

            if u % 2 == 0:
                @pl.when(grp >= 3)
                def _():
                    wait_group(aslot)

            reduce_batch(slot, aslot, (u % 2) * G)

            @pl.when(b + 3 < NB)
            def _():
                fire(b + 3, slot)
            if u % 2 == 1:
                write_group(grp, aslot)

        return carry

    lax.fori_loop(0, NB // 6, body, 0)

    drain(NB - 2, 0)
    wait_group(0)
    reduce_batch(0, 0, 0)
    drain(NB - 1, 1)
    reduce_batch(1, 0, G)
    write_group(NB // 2 - 1, 0)
    for aslot in range(3):
        wait_group(aslot)


@jax.jit
def kernel(tokens, table):
    tabT = table.T
    tail32 = jnp.pad(table[NBLK * 128:, :], ((0, 0), (0, DP - D)))
    idx = (
        tokens.astype(jnp.int32)
        .reshape(NUM_CHUNKS, K, B)
        .transpose(0, 2, 1)
        .reshape(NW, NB, G * K)
    )
    idx = jnp.pad(idx, ((0, 0), (0, 0), (0, 128 - G * K)))

    mesh = plsc.VectorSubcoreMesh(core_axis_name="c", subcore_axis_name="s")
    cp = pltpu.CompilerParams(use_tc_tiling_on_sc=True, needs_layout_passes=False)

    tabp = pl.kernel(
        _transpose_kernel,
        out_type=jax.ShapeDtypeStruct((VOCAB, DP), jnp.float32),
        mesh=mesh,
        scratch_types=[
            pltpu.VMEM((D, 128), jnp.float32),
            pltpu.VMEM((D, 128), jnp.float32),
        ] + [pltpu.VMEM((32, DP), jnp.float32)] * 2 + [
            pltpu.SemaphoreType.DMA,
            pltpu.SemaphoreType.DMA,
            pltpu.SemaphoreType.DMA,
            pltpu.SemaphoreType.DMA,
        ],
        compiler_params=cp,
    )(tabT, tail32)

    out = pl.kernel(
        _gather_kernel,
        out_type=jax.ShapeDtypeStruct((R, DP), jnp.float32),
        mesh=mesh,
        scratch_types=[pltpu.VMEM((NB, 128), jnp.int32)]
        + [pltpu.VMEM((G * K, 256), jnp.float32),
           pltpu.VMEM((G * K, 128), jnp.float32)] * 3
        + [pltpu.VMEM((8, DP), jnp.float32)] * 3
        + [pltpu.SemaphoreType.DMA] * 6,
        compiler_params=cp,
    )(idx, tabp)

    return out[:, :D].reshape(NUM_CHUNKS, B, D)
